# Initial kernel scaffold; baseline (speedup 1.0000x reference)
import jax
import jax.numpy as jnp
from jax.experimental import pallas as pl


def kernel(x, weight, bias):
    raise NotImplementedError("write your pallas kernel here")


# speedup vs baseline: 1.1295x; 1.1295x over previous
import functools

import jax
import jax.numpy as jnp
from jax.experimental import pallas as pl
from jax.experimental.pallas import tpu as pltpu

B, L, H, D, S = 4, 2048, 16, 128, 64
T = B * L
TB = 1024
NT = T // TB


def _router_body(x_ref, w_ref, b_ref, ones_ref, idx_ref, loss_ref,
                 sums_ref, cnts_ref):
    t = pl.program_id(0)

    @pl.when(t == 0)
    def _init():
        sums_ref[...] = jnp.zeros_like(sums_ref)
        cnts_ref[...] = jnp.zeros_like(cnts_ref)

    ones_ref[...] = jnp.ones_like(ones_ref)

    iota = jax.lax.broadcasted_iota(jnp.int32, (TB, S), 1)
    idx_cols = []
    for h in range(H):
        xh = x_ref[:, h * D:(h + 1) * D]
        logits = jnp.dot(xh, w_ref[h],
                         preferred_element_type=jnp.float32)
        logits = logits + b_ref[h][None, :]
        m = jnp.max(logits, axis=1, keepdims=True)
        e = jnp.exp(logits - m)
        denom = jnp.sum(e, axis=1, keepdims=True)
        score_sum = jnp.sum(e / denom, axis=0)
        idx = jnp.min(jnp.where(logits == m, iota, S), axis=1)
        idx_cols.append(idx[:, None])
        onehot = (iota == idx[:, None]).astype(jnp.float32)
        cnt = jnp.sum(onehot, axis=0)
        sums_ref[h, :] = sums_ref[h, :] + score_sum
        cnts_ref[h, :] = cnts_ref[h, :] + cnt

    idx_ref[...] = jnp.concatenate(idx_cols, axis=1)

    @pl.when(t == pl.num_programs(0) - 1)
    def _finish():
        prod = sums_ref[...] * cnts_ref[...]
        loss_ref[...] = (float(S) / (T * T)) * jnp.sum(prod, keepdims=True).reshape(1, 1)


@functools.partial(jax.jit, static_argnames=())
def kernel(x, weight, bias):
    dtype = x.dtype
    x2 = x.reshape(T, H * D)
    wt = jnp.transpose(weight.astype(jnp.float32), (0, 2, 1))

    ones_out, idx_out, loss_out = pl.pallas_call(
        _router_body,
        grid=(NT,),
        in_specs=[
            pl.BlockSpec((TB, H * D), lambda t: (t, 0)),
            pl.BlockSpec((H, D, S), lambda t: (0, 0, 0)),
            pl.BlockSpec((H, S), lambda t: (0, 0)),
        ],
        out_specs=[
            pl.BlockSpec((TB, H), lambda t: (t, 0)),
            pl.BlockSpec((TB, H), lambda t: (t, 0)),
            pl.BlockSpec((1, 1), lambda t: (0, 0)),
        ],
        out_shape=[
            jax.ShapeDtypeStruct((T, H), jnp.float32),
            jax.ShapeDtypeStruct((T, H), jnp.int32),
            jax.ShapeDtypeStruct((1, 1), jnp.float32),
        ],
        scratch_shapes=[
            pltpu.VMEM((H, S), jnp.float32),
            pltpu.VMEM((H, S), jnp.float32),
        ],
        compiler_params=pltpu.CompilerParams(
            dimension_semantics=("arbitrary",),
        ),
    )(x2.astype(jnp.float32), wt, bias.astype(jnp.float32))

    sg = ones_out.reshape(B, L, H).astype(dtype)
    idx = idx_out.reshape(B, L, H)
    loss = loss_out[0, 0].astype(dtype)
    return (sg, idx, loss)


# speedup vs baseline: 1.5161x; 1.3423x over previous
import functools

import jax
import jax.numpy as jnp
from jax.experimental import pallas as pl
from jax.experimental.pallas import tpu as pltpu

B, L, H, D, S = 4, 2048, 16, 128, 64
T = B * L
TB = 1024
NT = T // TB


def _router_body(x_ref, w_ref, b_ref, ones_ref, idx_ref, loss_ref,
                 sums_ref, cnts_ref):
    t = pl.program_id(0)

    @pl.when(t == 0)
    def _init():
        sums_ref[...] = jnp.zeros_like(sums_ref)
        cnts_ref[...] = jnp.zeros_like(cnts_ref)

    ones_ref[...] = jnp.ones_like(ones_ref)

    iota = jax.lax.broadcasted_iota(jnp.int32, (TB, S), 1)
    idx_cols = []
    for h in range(H):
        xh = x_ref[:, h, :]
        logits = jnp.dot(xh, w_ref[h],
                         preferred_element_type=jnp.float32)
        logits = logits + b_ref[h][None, :]
        m = jnp.max(logits, axis=1, keepdims=True)
        e = jnp.exp(logits - m)
        denom = jnp.sum(e, axis=1, keepdims=True)
        score_sum = jnp.sum(e / denom, axis=0)
        idx = jnp.min(jnp.where(logits == m, iota, S), axis=1)
        idx_cols.append(idx[:, None])
        onehot = (iota == idx[:, None]).astype(jnp.float32)
        cnt = jnp.sum(onehot, axis=0)
        sums_ref[h, :] = sums_ref[h, :] + score_sum
        cnts_ref[h, :] = cnts_ref[h, :] + cnt

    idx_ref[...] = jnp.concatenate(idx_cols, axis=1)

    @pl.when(t == pl.num_programs(0) - 1)
    def _finish():
        prod = sums_ref[...] * cnts_ref[...]
        loss_ref[...] = (float(S) / (T * T)) * jnp.sum(prod, keepdims=True).reshape(1, 1)


@functools.partial(jax.jit, static_argnames=())
def kernel(x, weight, bias):
    dtype = x.dtype
    x3 = x.reshape(T, H, D)
    wt = jnp.transpose(weight.astype(jnp.float32), (0, 2, 1))

    ones_out, idx_out, loss_out = pl.pallas_call(
        _router_body,
        grid=(NT,),
        in_specs=[
            pl.BlockSpec((TB, H, D), lambda t: (t, 0, 0)),
            pl.BlockSpec((H, D, S), lambda t: (0, 0, 0)),
            pl.BlockSpec((H, S), lambda t: (0, 0)),
        ],
        out_specs=[
            pl.BlockSpec((TB, H), lambda t: (t, 0)),
            pl.BlockSpec((TB, H), lambda t: (t, 0)),
            pl.BlockSpec((1, 1), lambda t: (0, 0)),
        ],
        out_shape=[
            jax.ShapeDtypeStruct((T, H), jnp.float32),
            jax.ShapeDtypeStruct((T, H), jnp.int32),
            jax.ShapeDtypeStruct((1, 1), jnp.float32),
        ],
        scratch_shapes=[
            pltpu.VMEM((H, S), jnp.float32),
            pltpu.VMEM((H, S), jnp.float32),
        ],
        compiler_params=pltpu.CompilerParams(
            dimension_semantics=("arbitrary",),
        ),
    )(x3.astype(jnp.float32), wt, bias.astype(jnp.float32))

    sg = ones_out.reshape(B, L, H).astype(dtype)
    idx = idx_out.reshape(B, L, H)
    loss = loss_out[0, 0].astype(dtype)
    return (sg, idx, loss)
